# Initial kernel scaffold; baseline (speedup 1.0000x reference)
import jax
import jax.numpy as jnp
from jax.experimental import pallas as pl


def kernel(idx, table):
    raise NotImplementedError("write your pallas kernel here")


# speedup vs baseline: 1.0294x; 1.0294x over previous
import functools

import jax
import jax.numpy as jnp
from jax import lax
from jax.experimental import pallas as pl
from jax.experimental.pallas import tpu as pltpu
from jax.experimental.pallas import tpu_sc as plsc

_NC = 2
_NS = 16
_NW = _NC * _NS
_CHUNK = 128


def kernel(idx, table):
    B = idx.shape[0]
    n_chunks = B // (_NW * _CHUNK)
    per_w = n_chunks * _CHUNK

    idx_r = idx.reshape(_NW, n_chunks, _CHUNK)

    mesh = plsc.VectorSubcoreMesh(
        core_axis_name="c", subcore_axis_name="s",
        num_cores=_NC, num_subcores=_NS,
    )

    @functools.partial(
        pl.kernel,
        out_type=jax.ShapeDtypeStruct((_NW, per_w), jnp.int32),
        mesh=mesh,
        scratch_types=[
            pltpu.VMEM((n_chunks, _CHUNK), jnp.int32),
            pltpu.VMEM((per_w,), jnp.int32),
            pltpu.SemaphoreType.DMA,
        ],
    )
    def gather_kernel(table_hbm, idx_hbm, out_hbm, idx_v, vals_v, sem):
        wid = lax.axis_index("s") * _NC + lax.axis_index("c")
        pltpu.sync_copy(idx_hbm.at[wid], idx_v)
        copies = [
            pltpu.async_copy(
                table_hbm.at[idx_v.at[j]],
                vals_v.at[pl.ds(j * _CHUNK, _CHUNK)],
                sem,
            )
            for j in range(n_chunks)
        ]
        for c in copies:
            c.wait()
        pltpu.sync_copy(vals_v, out_hbm.at[wid])

    out = gather_kernel(table, idx_r)
    return out.reshape(B)


# speedup vs baseline: 1.0336x; 1.0041x over previous
import functools

import jax
import jax.numpy as jnp
from jax import lax
from jax.experimental import pallas as pl
from jax.experimental.pallas import tpu as pltpu
from jax.experimental.pallas import tpu_sc as plsc

_NC = 2
_NS = 16
_NW = _NC * _NS
_CHUNK = 128


def kernel(idx, table):
    B = idx.shape[0]
    n_chunks = B // (_NW * _CHUNK)
    per_w = n_chunks * _CHUNK

    idx_r = idx.reshape(_NW, per_w)

    mesh = plsc.VectorSubcoreMesh(
        core_axis_name="c", subcore_axis_name="s",
        num_cores=_NC, num_subcores=_NS,
    )

    @functools.partial(
        pl.kernel,
        out_type=jax.ShapeDtypeStruct((_NW, per_w), jnp.int32),
        mesh=mesh,
        scratch_types=[
            pltpu.VMEM((per_w,), jnp.int32),
            pltpu.VMEM((per_w,), jnp.int32),
            pltpu.SemaphoreType.DMA,
        ],
    )
    def gather_kernel(table_hbm, idx_hbm, out_hbm, idx_v, vals_v, sem):
        wid = lax.axis_index("s") * _NC + lax.axis_index("c")
        pltpu.sync_copy(idx_hbm.at[wid], idx_v)
        pltpu.async_copy(table_hbm.at[idx_v], vals_v, sem).wait()
        pltpu.sync_copy(vals_v, out_hbm.at[wid])

    out = gather_kernel(table, idx_r)
    return out.reshape(B)


# speedup vs baseline: 1.0425x; 1.0086x over previous
import functools

import jax
import jax.numpy as jnp
from jax import lax
from jax.experimental import pallas as pl
from jax.experimental.pallas import tpu as pltpu
from jax.experimental.pallas import tpu_sc as plsc

_NC = 2
_NS = 16
_NW = _NC * _NS
_CHUNK = 128


def kernel(idx, table):
    B = idx.shape[0]
    n_chunks = B // (_NW * _CHUNK)
    per_w = n_chunks * _CHUNK

    idx_r = idx.reshape(_NW, per_w)

    mesh = plsc.VectorSubcoreMesh(
        core_axis_name="c", subcore_axis_name="s",
        num_cores=_NC, num_subcores=_NS,
    )

    @functools.partial(
        pl.kernel,
        out_type=jax.ShapeDtypeStruct((_NW, per_w), jnp.int32),
        mesh=mesh,
        scratch_types=[
            pltpu.VMEM((per_w,), jnp.int32),
            pltpu.VMEM((per_w,), jnp.int32),
            pltpu.SemaphoreType.DMA,
            pltpu.SemaphoreType.DMA,
            pltpu.SemaphoreType.DMA,
            pltpu.SemaphoreType.DMA,
            pltpu.SemaphoreType.DMA,
        ],
    )
    def gather_kernel(table_hbm, idx_hbm, out_hbm, idx_v, vals_v,
                      sem_i0, sem_i1, sem_g0, sem_g1, sem_o):
        wid = lax.axis_index("s") * _NC + lax.axis_index("c")
        half = per_w // 2
        lo = pl.ds(0, half)
        hi = pl.ds(half, half)
        ci0 = pltpu.async_copy(idx_hbm.at[wid, lo], idx_v.at[lo], sem_i0)
        ci1 = pltpu.async_copy(idx_hbm.at[wid, hi], idx_v.at[hi], sem_i1)
        ci0.wait()
        g0 = pltpu.async_copy(table_hbm.at[idx_v.at[lo]], vals_v.at[lo], sem_g0)
        ci1.wait()
        g1 = pltpu.async_copy(table_hbm.at[idx_v.at[hi]], vals_v.at[hi], sem_g1)
        g0.wait()
        o0 = pltpu.async_copy(vals_v.at[lo], out_hbm.at[wid, lo], sem_o)
        g1.wait()
        o1 = pltpu.async_copy(vals_v.at[hi], out_hbm.at[wid, hi], sem_o)
        o0.wait()
        o1.wait()

    out = gather_kernel(table, idx_r)
    return out.reshape(B)


# speedup vs baseline: 1.0433x; 1.0008x over previous
import functools

import jax
import jax.numpy as jnp
from jax import lax
from jax.experimental import pallas as pl
from jax.experimental.pallas import tpu as pltpu
from jax.experimental.pallas import tpu_sc as plsc

_NC = 2
_NS = 16
_NW = _NC * _NS
_CHUNK = 128
_NP = 4


def kernel(idx, table):
    B = idx.shape[0]
    n_chunks = B // (_NW * _CHUNK)
    per_w = n_chunks * _CHUNK

    idx_r = idx.reshape(_NW, per_w)

    mesh = plsc.VectorSubcoreMesh(
        core_axis_name="c", subcore_axis_name="s",
        num_cores=_NC, num_subcores=_NS,
    )

    @functools.partial(
        pl.kernel,
        out_type=jax.ShapeDtypeStruct((_NW, per_w), jnp.int32),
        mesh=mesh,
        scratch_types=[
            pltpu.VMEM((per_w,), jnp.int32),
            pltpu.VMEM((per_w,), jnp.int32),
            pltpu.SemaphoreType.DMA((_NP,)),
            pltpu.SemaphoreType.DMA((_NP,)),
            pltpu.SemaphoreType.DMA,
        ],
    )
    def gather_kernel(table_hbm, idx_hbm, out_hbm, idx_v, vals_v,
                      sem_i, sem_g, sem_o):
        wid = lax.axis_index("s") * _NC + lax.axis_index("c")
        q = per_w // _NP
        sl = [pl.ds(j * q, q) for j in range(_NP)]
        ci = [
            pltpu.async_copy(idx_hbm.at[wid, sl[j]], idx_v.at[sl[j]],
                             sem_i.at[j])
            for j in range(_NP)
        ]
        gs = []
        for j in range(_NP):
            ci[j].wait()
            gs.append(
                pltpu.async_copy(table_hbm.at[idx_v.at[sl[j]]],
                                 vals_v.at[sl[j]], sem_g.at[j])
            )
        os = []
        for j in range(_NP):
            gs[j].wait()
            os.append(
                pltpu.async_copy(vals_v.at[sl[j]], out_hbm.at[wid, sl[j]],
                                 sem_o)
            )
        for o in os:
            o.wait()

    out = gather_kernel(table, idx_r)
    return out.reshape(B)
